# baseline (speedup 1.0000x reference)
import functools

import jax
import jax.numpy as jnp
from jax import lax
from jax.experimental import pallas as pl
from jax.experimental.pallas import tpu as pltpu
from jax.experimental.pallas import tpu_sc as plsc

N = 10000
E = 320000
D = 128
NPAD = 10240
RPT = NPAD // 16
NW = 32
EPT = E // NW
K = 80
NCH = EPT // K
EPTP = 10080
PADE = EPTP - EPT
K2 = 112
NCH2 = EPTP // K2
NB = 2

_mesh = plsc.VectorSubcoreMesh(core_axis_name="c", subcore_axis_name="s")


def _zero_1d(ref, n):
    def zb(i, c):
        ref[pl.ds(i * 16, 16)] = jnp.zeros((16,), jnp.float32)
        return c

    lax.fori_loop(0, n // 16, zb, 0)


def _sc_hist_body(tab_hbm, row_hbm, col_hbm, out_hbm, shared, use_tab):
    cid = lax.axis_index("c")
    sid = lax.axis_index("s")
    wid = sid * 2 + cid

    def _scoped(rowbuf, colbuf, tabv, accv, stage, redacc, lsem):
        cps = [pltpu.async_copy(col_hbm.at[wid], colbuf, lsem)]
        if use_tab:
            cps.append(pltpu.async_copy(tab_hbm, tabv, lsem))
            cps.append(pltpu.async_copy(row_hbm.at[wid], rowbuf, lsem))
        _zero_1d(accv, NPAD)
        for cp in cps:
            cp.wait()
        ones16 = jnp.full((16,), 1.0, jnp.float32)

        def body(j, c):
            c16 = colbuf[pl.ds(j * 16, 16)]
            if use_tab:
                r16 = rowbuf[pl.ds(j * 16, 16)]
                vals = plsc.load_gather(tabv, [r16])
            else:
                vals = ones16
            plsc.addupdate_scatter(accv, [c16], vals)
            return c

        lax.fori_loop(0, EPT // 16, body, 0)
        pltpu.sync_copy(accv, shared.at[sid])
        plsc.subcore_barrier()
        pltpu.sync_copy(shared.at[:, pl.ds(sid * RPT, RPT)], stage)

        def add_i(i, c2):
            acc16 = stage[0, pl.ds(i * 16, 16)]
            for t in range(1, 16):
                acc16 = acc16 + stage[t, pl.ds(i * 16, 16)]
            redacc[pl.ds(i * 16, 16)] = acc16
            return c2

        lax.fori_loop(0, RPT // 16, add_i, 0)
        pltpu.sync_copy(redacc, out_hbm.at[cid, pl.ds(sid * RPT, RPT)])

    pl.run_scoped(
        _scoped,
        pltpu.VMEM((EPT,), jnp.int32),
        pltpu.VMEM((EPT,), jnp.int32),
        pltpu.VMEM((N,), jnp.float32),
        pltpu.VMEM((NPAD,), jnp.float32),
        pltpu.VMEM((16, RPT), jnp.float32),
        pltpu.VMEM((RPT,), jnp.float32),
        pltpu.SemaphoreType.DMA,
    )


@functools.partial(
    pl.kernel,
    out_type=jax.ShapeDtypeStruct((2, NPAD), jnp.float32),
    mesh=_mesh,
    compiler_params=pltpu.CompilerParams(needs_layout_passes=False),
    scratch_types=[pltpu.VMEM_SHARED((16, NPAD), jnp.float32)],
)
def _sc_degree(col_hbm, dummy_tab, dummy_row, out_hbm, shared):
    _sc_hist_body(dummy_tab, dummy_row, col_hbm, out_hbm, shared, use_tab=False)


@functools.partial(
    pl.kernel,
    out_type=jax.ShapeDtypeStruct((2, NPAD), jnp.float32),
    mesh=_mesh,
    compiler_params=pltpu.CompilerParams(needs_layout_passes=False),
    scratch_types=[pltpu.VMEM_SHARED((16, NPAD), jnp.float32)],
)
def _sc_agg1(tab_hbm, row_hbm, col_hbm, out_hbm, shared):
    _sc_hist_body(tab_hbm, row_hbm, col_hbm, out_hbm, shared, use_tab=True)


@functools.partial(
    pl.kernel,
    out_type=jax.ShapeDtypeStruct((2, NPAD, D), jnp.float32),
    mesh=_mesh,
    compiler_params=pltpu.CompilerParams(use_tc_tiling_on_sc=False),
    scratch_types=[pltpu.VMEM_SHARED((NPAD, D), jnp.float32)],
)
def _sc_agg128(tab_hbm, row_hbm, col_hbm, out_hbm, shared):
    cid = lax.axis_index("c")
    sid = lax.axis_index("s")
    wid = sid * 2 + cid

    def _scoped(rowbuf, colbuf, gbufs, sems, lsem):
        cps = [
            pltpu.async_copy(row_hbm.at[wid], rowbuf, lsem),
            pltpu.async_copy(col_hbm.at[wid], colbuf, lsem),
        ]
        def zb(i, c):
            for cc in range(D // 16):
                gbufs[0][i, pl.ds(cc * 16, 16)] = jnp.zeros((16,), jnp.float32)
            return c

        lax.fori_loop(0, K2, zb, 0)
        zoff = 0
        while zoff < RPT:
            zn = min(K2, RPT - zoff)
            pltpu.sync_copy(
                gbufs[0].at[pl.ds(0, zn)],
                shared.at[pl.ds(sid * RPT + zoff, zn)],
            )
            zoff += zn
        for cp in cps:
            cp.wait()
        plsc.subcore_barrier()

        def _gidx(j):
            return rowbuf.at[j]

        for b in range(NB):
            pltpu.async_copy(tab_hbm.at[_gidx(b)], gbufs[b], sems[b])

        def body(m, c):
            for b in range(NB):
                j = m * NB + b
                pltpu.make_async_copy(
                    tab_hbm.at[_gidx(j)], gbufs[b], sems[b]
                ).wait()
                pltpu.sync_copy(gbufs[b], shared.at[colbuf.at[j]], add=True)

                @pl.when(j + NB < NCH2)
                def _():
                    pltpu.async_copy(tab_hbm.at[_gidx(j + NB)], gbufs[b], sems[b])

            return c

        lax.fori_loop(0, NCH2 // NB, body, 0)
        for jt in range(NCH2 // NB * NB, NCH2):
            b = jt % NB
            pltpu.make_async_copy(tab_hbm.at[_gidx(jt)], gbufs[b], sems[b]).wait()
            pltpu.sync_copy(gbufs[b], shared.at[colbuf.at[jt]], add=True)
        plsc.subcore_barrier()
        pltpu.sync_copy(
            shared.at[pl.ds(sid * RPT, RPT)],
            out_hbm.at[cid, pl.ds(sid * RPT, RPT)],
        )

    pl.run_scoped(
        _scoped,
        pltpu.VMEM((NCH2, K2), jnp.int32),
        pltpu.VMEM((NCH2, K2), jnp.int32),
        [pltpu.VMEM((K2, D), jnp.float32) for _ in range(NB)],
        [pltpu.SemaphoreType.DMA for _ in range(NB)],
        pltpu.SemaphoreType.DMA,
    )


def _tc_mm_body(x_ref, w1_ref, h1_ref):
    h1_ref[...] = jnp.dot(
        x_ref[...], w1_ref[...], preferred_element_type=jnp.float32
    )


_tc_mm = pl.pallas_call(
    _tc_mm_body,
    out_shape=jax.ShapeDtypeStruct((N, D), jnp.float32),
)


def _tc1_body(degp_ref, h1_ref, h1s_ref, dis_ref):
    deg = degp_ref[0, :N] + degp_ref[1, :N] + 1.0
    dis = lax.rsqrt(deg)
    dis_ref[...] = dis
    h1s_ref[...] = h1_ref[...] * dis[:, None]


_tc1 = pl.pallas_call(
    _tc1_body,
    out_shape=[
        jax.ShapeDtypeStruct((N, D), jnp.float32),
        jax.ShapeDtypeStruct((N,), jnp.float32),
    ],
)


def _tc2_body(p_ref, h1s_ref, dis_ref, b1_ref, w2_ref, h2s_ref):
    dis = dis_ref[...]
    acc = p_ref[0, :N, :] + p_ref[1, :N, :] + h1s_ref[...]
    h = jnp.maximum(acc * dis[:, None] + b1_ref[...][None, :], 0.0)
    h2 = jnp.dot(h, w2_ref[...], preferred_element_type=jnp.float32)
    h2s_ref[...] = h2[:, 0] * dis


_tc2 = pl.pallas_call(
    _tc2_body,
    out_shape=jax.ShapeDtypeStruct((N,), jnp.float32),
)


def _tc3_body(q_ref, h2s_ref, dis_ref, b2_ref, out_ref):
    out_ref[...] = (
        dis_ref[...] * (q_ref[0, :N] + q_ref[1, :N] + h2s_ref[...]) + b2_ref[...]
    )


_tc3 = pl.pallas_call(
    _tc3_body,
    out_shape=jax.ShapeDtypeStruct((N,), jnp.float32),
)


def kernel(x, edge_index, W1, b1, W2, b2):
    row2 = edge_index[0].reshape(NW, EPT)
    col2 = edge_index[1].reshape(NW, EPT)
    row3 = jnp.concatenate(
        [row2, jnp.zeros((NW, PADE), jnp.int32)], axis=1
    ).reshape(NW, NCH2, K2)
    col3 = jnp.concatenate(
        [col2, jnp.full((NW, PADE), N, jnp.int32)], axis=1
    ).reshape(NW, NCH2, K2)
    dummy_tab = jnp.zeros((N,), jnp.float32)

    degp = _sc_degree(col2, dummy_tab, row2)
    h1 = _tc_mm(x, W1)
    h1s, dis = _tc1(degp, h1)
    p = _sc_agg128(h1s, row3, col3)
    h2s = _tc2(p, h1s, dis, b1, W2)
    q = _sc_agg1(h2s, row2, col2)
    out = _tc3(q, h2s, dis, b2)
    return out[:, None]


# speedup vs baseline: 1.4369x; 1.4369x over previous
import functools

import jax
import jax.numpy as jnp
from jax import lax
from jax.experimental import pallas as pl
from jax.experimental.pallas import tpu as pltpu
from jax.experimental.pallas import tpu_sc as plsc

N = 10000
E = 320000
D = 128
NPAD = 10240
RPT = NPAD // 16
NW = 32
EPT = E // NW
K = 80
NCH = EPT // K
EPTP = 10000
PADE = EPTP - EPT
K2 = 80
NCH2 = EPTP // K2
NB = 2

_mesh = plsc.VectorSubcoreMesh(core_axis_name="c", subcore_axis_name="s")


def _zero_1d(ref, n):
    def zb(i, c):
        ref[pl.ds(i * 16, 16)] = jnp.zeros((16,), jnp.float32)
        return c

    lax.fori_loop(0, n // 16, zb, 0)


def _sc_hist_body(tab_hbm, row_hbm, col_hbm, out_hbm, shared, use_tab):
    cid = lax.axis_index("c")
    sid = lax.axis_index("s")
    wid = sid * 2 + cid

    def _scoped(rowbuf, colbuf, tabv, accv, stage, redacc, lsem):
        cps = [pltpu.async_copy(col_hbm.at[wid], colbuf, lsem)]
        if use_tab:
            cps.append(pltpu.async_copy(tab_hbm, tabv, lsem))
            cps.append(pltpu.async_copy(row_hbm.at[wid], rowbuf, lsem))
        _zero_1d(accv, NPAD)
        for cp in cps:
            cp.wait()
        ones16 = jnp.full((16,), 1.0, jnp.float32)

        def body(j, c):
            c16 = colbuf[pl.ds(j * 16, 16)]
            if use_tab:
                r16 = rowbuf[pl.ds(j * 16, 16)]
                vals = plsc.load_gather(tabv, [r16])
            else:
                vals = ones16
            plsc.addupdate_scatter(accv, [c16], vals)
            return c

        lax.fori_loop(0, EPT // 16, body, 0)
        pltpu.sync_copy(accv, shared.at[sid])
        plsc.subcore_barrier()
        pltpu.sync_copy(shared.at[:, pl.ds(sid * RPT, RPT)], stage)

        def add_i(i, c2):
            acc16 = stage[0, pl.ds(i * 16, 16)]
            for t in range(1, 16):
                acc16 = acc16 + stage[t, pl.ds(i * 16, 16)]
            redacc[pl.ds(i * 16, 16)] = acc16
            return c2

        lax.fori_loop(0, RPT // 16, add_i, 0)
        pltpu.sync_copy(redacc, out_hbm.at[cid, pl.ds(sid * RPT, RPT)])

    pl.run_scoped(
        _scoped,
        pltpu.VMEM((EPT,), jnp.int32),
        pltpu.VMEM((EPT,), jnp.int32),
        pltpu.VMEM((N,), jnp.float32),
        pltpu.VMEM((NPAD,), jnp.float32),
        pltpu.VMEM((16, RPT), jnp.float32),
        pltpu.VMEM((RPT,), jnp.float32),
        pltpu.SemaphoreType.DMA,
    )


@functools.partial(
    pl.kernel,
    out_type=jax.ShapeDtypeStruct((2, NPAD), jnp.float32),
    mesh=_mesh,
    compiler_params=pltpu.CompilerParams(needs_layout_passes=False),
    scratch_types=[pltpu.VMEM_SHARED((16, NPAD), jnp.float32)],
)
def _sc_degree(col_hbm, dummy_tab, dummy_row, out_hbm, shared):
    _sc_hist_body(dummy_tab, dummy_row, col_hbm, out_hbm, shared, use_tab=False)


@functools.partial(
    pl.kernel,
    out_type=jax.ShapeDtypeStruct((2, NPAD), jnp.float32),
    mesh=_mesh,
    compiler_params=pltpu.CompilerParams(needs_layout_passes=False),
    scratch_types=[pltpu.VMEM_SHARED((16, NPAD), jnp.float32)],
)
def _sc_agg1(tab_hbm, row_hbm, col_hbm, out_hbm, shared):
    _sc_hist_body(tab_hbm, row_hbm, col_hbm, out_hbm, shared, use_tab=True)


@functools.partial(
    pl.kernel,
    out_type=jax.ShapeDtypeStruct((2, NPAD, D), jnp.float32),
    mesh=_mesh,
    compiler_params=pltpu.CompilerParams(use_tc_tiling_on_sc=False),
    scratch_types=[pltpu.VMEM_SHARED((NPAD, D), jnp.float32)],
)
def _sc_agg128(tab_hbm, row_hbm, col_hbm, out_hbm, shared):
    cid = lax.axis_index("c")
    sid = lax.axis_index("s")
    wid = sid * 2 + cid

    def _scoped(rowbuf, colbuf, gbufs, sems, lsem):
        cps = [
            pltpu.async_copy(row_hbm.at[wid], rowbuf, lsem),
            pltpu.async_copy(col_hbm.at[wid], colbuf, lsem),
        ]
        def zb(i, c):
            for cc in range(D // 16):
                gbufs[0][i, pl.ds(cc * 16, 16)] = jnp.zeros((16,), jnp.float32)
            return c

        lax.fori_loop(0, K2, zb, 0)
        zoff = 0
        while zoff < RPT:
            zn = min(K2, RPT - zoff)
            pltpu.sync_copy(
                gbufs[0].at[pl.ds(0, zn)],
                shared.at[pl.ds(sid * RPT + zoff, zn)],
            )
            zoff += zn
        for cp in cps:
            cp.wait()
        plsc.subcore_barrier()

        def _gidx(j):
            return rowbuf.at[j]

        for b in range(NB):
            pltpu.async_copy(tab_hbm.at[_gidx(b)], gbufs[b], sems[b])

        def body(m, c):
            for b in range(NB):
                j = m * NB + b
                pltpu.make_async_copy(
                    tab_hbm.at[_gidx(j)], gbufs[b], sems[b]
                ).wait()
                pltpu.sync_copy(gbufs[b], shared.at[colbuf.at[j]], add=True)

                @pl.when(j + NB < NCH2)
                def _():
                    pltpu.async_copy(tab_hbm.at[_gidx(j + NB)], gbufs[b], sems[b])

            return c

        lax.fori_loop(0, NCH2 // NB, body, 0)
        for jt in range(NCH2 // NB * NB, NCH2):
            b = jt % NB
            pltpu.make_async_copy(tab_hbm.at[_gidx(jt)], gbufs[b], sems[b]).wait()
            pltpu.sync_copy(gbufs[b], shared.at[colbuf.at[jt]], add=True)
        plsc.subcore_barrier()
        pltpu.sync_copy(
            shared.at[pl.ds(sid * RPT, RPT)],
            out_hbm.at[cid, pl.ds(sid * RPT, RPT)],
        )

    pl.run_scoped(
        _scoped,
        pltpu.VMEM((NCH2, K2), jnp.int32),
        pltpu.VMEM((NCH2, K2), jnp.int32),
        [pltpu.VMEM((K2, D), jnp.float32) for _ in range(NB)],
        [pltpu.SemaphoreType.DMA for _ in range(NB)],
        pltpu.SemaphoreType.DMA,
    )


def _tc_mm_body(x_ref, w1_ref, h1_ref):
    h1_ref[...] = jnp.dot(
        x_ref[...], w1_ref[...], preferred_element_type=jnp.float32
    )


_tc_mm = pl.pallas_call(
    _tc_mm_body,
    out_shape=jax.ShapeDtypeStruct((N, D), jnp.float32),
)


def _tc1_body(degp_ref, h1_ref, h1s_ref, dis_ref):
    deg = degp_ref[0, :N] + degp_ref[1, :N] + 1.0
    dis = lax.rsqrt(deg)
    dis_ref[...] = dis
    h1s_ref[...] = h1_ref[...] * dis[:, None]


_tc1 = pl.pallas_call(
    _tc1_body,
    out_shape=[
        jax.ShapeDtypeStruct((N, D), jnp.float32),
        jax.ShapeDtypeStruct((N,), jnp.float32),
    ],
)


def _tc2_body(p_ref, h1s_ref, dis_ref, b1_ref, w2_ref, h2s_ref):
    dis = dis_ref[...]
    acc = p_ref[0, :N, :] + p_ref[1, :N, :] + h1s_ref[...]
    h = jnp.maximum(acc * dis[:, None] + b1_ref[...][None, :], 0.0)
    h2 = jnp.dot(h, w2_ref[...], preferred_element_type=jnp.float32)
    h2s_ref[...] = h2[:, 0] * dis


_tc2 = pl.pallas_call(
    _tc2_body,
    out_shape=jax.ShapeDtypeStruct((N,), jnp.float32),
)


def _tc3_body(q_ref, h2s_ref, dis_ref, b2_ref, out_ref):
    out_ref[...] = (
        dis_ref[...] * (q_ref[0, :N] + q_ref[1, :N] + h2s_ref[...]) + b2_ref[...]
    )


_tc3 = pl.pallas_call(
    _tc3_body,
    out_shape=jax.ShapeDtypeStruct((N,), jnp.float32),
)


def kernel(x, edge_index, W1, b1, W2, b2):
    row2 = edge_index[0].reshape(NW, EPT)
    col2 = edge_index[1].reshape(NW, EPT)
    row3 = row2.reshape(NW, NCH2, K2)
    col3 = col2.reshape(NW, NCH2, K2)
    dummy_tab = jnp.zeros((N,), jnp.float32)

    degp = _sc_degree(col2, dummy_tab, row2)
    h1 = _tc_mm(x, W1)
    h1s, dis = _tc1(degp, h1)
    p = _sc_agg128(h1s, row3, col3)
    h2s = _tc2(p, h1s, dis, b1, W2)
    q = _sc_agg1(h2s, row2, col2)
    out = _tc3(q, h2s, dis, b2)
    return out[:, None]


# speedup vs baseline: 1.5998x; 1.1134x over previous
import functools

import jax
import jax.numpy as jnp
from jax import lax
from jax.experimental import pallas as pl
from jax.experimental.pallas import tpu as pltpu
from jax.experimental.pallas import tpu_sc as plsc

N = 10000
E = 320000
D = 128
NPAD = 10240
RPT = NPAD // 16
NW = 32
EPT = E // NW
K = 80
NCH = EPT // K
EPTP = 10000
PADE = EPTP - EPT
K2 = 40
NCH2 = EPTP // K2
NB = 4

_mesh = plsc.VectorSubcoreMesh(core_axis_name="c", subcore_axis_name="s")


def _zero_1d(ref, n):
    def zb(i, c):
        ref[pl.ds(i * 16, 16)] = jnp.zeros((16,), jnp.float32)
        return c

    lax.fori_loop(0, n // 16, zb, 0)


def _sc_hist_body(tab_hbm, row_hbm, col_hbm, out_hbm, shared, use_tab):
    cid = lax.axis_index("c")
    sid = lax.axis_index("s")
    wid = sid * 2 + cid

    def _scoped(rowbuf, colbuf, tabv, accv, stage, redacc, lsem):
        cps = [pltpu.async_copy(col_hbm.at[wid], colbuf, lsem)]
        if use_tab:
            cps.append(pltpu.async_copy(tab_hbm, tabv, lsem))
            cps.append(pltpu.async_copy(row_hbm.at[wid], rowbuf, lsem))
        _zero_1d(accv, NPAD)
        for cp in cps:
            cp.wait()
        ones16 = jnp.full((16,), 1.0, jnp.float32)

        def body(j, c):
            c16 = colbuf[pl.ds(j * 16, 16)]
            if use_tab:
                r16 = rowbuf[pl.ds(j * 16, 16)]
                vals = plsc.load_gather(tabv, [r16])
            else:
                vals = ones16
            plsc.addupdate_scatter(accv, [c16], vals)
            return c

        lax.fori_loop(0, EPT // 16, body, 0)
        pltpu.sync_copy(accv, shared.at[sid])
        plsc.subcore_barrier()
        pltpu.sync_copy(shared.at[:, pl.ds(sid * RPT, RPT)], stage)

        def add_i(i, c2):
            acc16 = stage[0, pl.ds(i * 16, 16)]
            for t in range(1, 16):
                acc16 = acc16 + stage[t, pl.ds(i * 16, 16)]
            redacc[pl.ds(i * 16, 16)] = acc16
            return c2

        lax.fori_loop(0, RPT // 16, add_i, 0)
        pltpu.sync_copy(redacc, out_hbm.at[cid, pl.ds(sid * RPT, RPT)])

    pl.run_scoped(
        _scoped,
        pltpu.VMEM((EPT,), jnp.int32),
        pltpu.VMEM((EPT,), jnp.int32),
        pltpu.VMEM((N,), jnp.float32),
        pltpu.VMEM((NPAD,), jnp.float32),
        pltpu.VMEM((16, RPT), jnp.float32),
        pltpu.VMEM((RPT,), jnp.float32),
        pltpu.SemaphoreType.DMA,
    )


@functools.partial(
    pl.kernel,
    out_type=jax.ShapeDtypeStruct((2, NPAD), jnp.float32),
    mesh=_mesh,
    compiler_params=pltpu.CompilerParams(needs_layout_passes=False),
    scratch_types=[pltpu.VMEM_SHARED((16, NPAD), jnp.float32)],
)
def _sc_degree(col_hbm, dummy_tab, dummy_row, out_hbm, shared):
    _sc_hist_body(dummy_tab, dummy_row, col_hbm, out_hbm, shared, use_tab=False)


@functools.partial(
    pl.kernel,
    out_type=jax.ShapeDtypeStruct((2, NPAD), jnp.float32),
    mesh=_mesh,
    compiler_params=pltpu.CompilerParams(needs_layout_passes=False),
    scratch_types=[pltpu.VMEM_SHARED((16, NPAD), jnp.float32)],
)
def _sc_agg1(tab_hbm, row_hbm, col_hbm, out_hbm, shared):
    _sc_hist_body(tab_hbm, row_hbm, col_hbm, out_hbm, shared, use_tab=True)


@functools.partial(
    pl.kernel,
    out_type=jax.ShapeDtypeStruct((2, NPAD, D), jnp.float32),
    mesh=_mesh,
    compiler_params=pltpu.CompilerParams(use_tc_tiling_on_sc=False),
    scratch_types=[pltpu.VMEM_SHARED((NPAD, D), jnp.float32)],
)
def _sc_agg128(tab_hbm, row_hbm, col_hbm, out_hbm, shared):
    cid = lax.axis_index("c")
    sid = lax.axis_index("s")
    wid = sid * 2 + cid

    def _scoped(rowbuf, colbuf, gbufs, sems, lsem):
        cps = [
            pltpu.async_copy(row_hbm.at[wid], rowbuf, lsem),
            pltpu.async_copy(col_hbm.at[wid], colbuf, lsem),
        ]
        def zb(i, c):
            for cc in range(D // 16):
                gbufs[0][i, pl.ds(cc * 16, 16)] = jnp.zeros((16,), jnp.float32)
            return c

        lax.fori_loop(0, K2, zb, 0)
        zoff = 0
        while zoff < RPT:
            zn = min(K2, RPT - zoff)
            pltpu.sync_copy(
                gbufs[0].at[pl.ds(0, zn)],
                shared.at[pl.ds(sid * RPT + zoff, zn)],
            )
            zoff += zn
        for cp in cps:
            cp.wait()
        plsc.subcore_barrier()

        def _gidx(j):
            return rowbuf.at[j]

        for b in range(NB):
            pltpu.async_copy(tab_hbm.at[_gidx(b)], gbufs[b], sems[b])

        def body(m, c):
            for b in range(NB):
                j = m * NB + b
                pltpu.make_async_copy(
                    tab_hbm.at[_gidx(j)], gbufs[b], sems[b]
                ).wait()
                pltpu.sync_copy(gbufs[b], shared.at[colbuf.at[j]], add=True)

                @pl.when(j + NB < NCH2)
                def _():
                    pltpu.async_copy(tab_hbm.at[_gidx(j + NB)], gbufs[b], sems[b])

            return c

        lax.fori_loop(0, NCH2 // NB, body, 0)
        for jt in range(NCH2 // NB * NB, NCH2):
            b = jt % NB
            pltpu.make_async_copy(tab_hbm.at[_gidx(jt)], gbufs[b], sems[b]).wait()
            pltpu.sync_copy(gbufs[b], shared.at[colbuf.at[jt]], add=True)
        plsc.subcore_barrier()
        pltpu.sync_copy(
            shared.at[pl.ds(sid * RPT, RPT)],
            out_hbm.at[cid, pl.ds(sid * RPT, RPT)],
        )

    pl.run_scoped(
        _scoped,
        pltpu.VMEM((NCH2, K2), jnp.int32),
        pltpu.VMEM((NCH2, K2), jnp.int32),
        [pltpu.VMEM((K2, D), jnp.float32) for _ in range(NB)],
        [pltpu.SemaphoreType.DMA for _ in range(NB)],
        pltpu.SemaphoreType.DMA,
    )


def _tc_mm_body(x_ref, w1_ref, h1_ref):
    h1_ref[...] = jnp.dot(
        x_ref[...], w1_ref[...], preferred_element_type=jnp.float32
    )


_tc_mm = pl.pallas_call(
    _tc_mm_body,
    out_shape=jax.ShapeDtypeStruct((N, D), jnp.float32),
)


def _tc1_body(degp_ref, h1_ref, h1s_ref, dis_ref):
    deg = degp_ref[0, :N] + degp_ref[1, :N] + 1.0
    dis = lax.rsqrt(deg)
    dis_ref[...] = dis
    h1s_ref[...] = h1_ref[...] * dis[:, None]


_tc1 = pl.pallas_call(
    _tc1_body,
    out_shape=[
        jax.ShapeDtypeStruct((N, D), jnp.float32),
        jax.ShapeDtypeStruct((N,), jnp.float32),
    ],
)


def _tc2_body(p_ref, h1s_ref, dis_ref, b1_ref, w2_ref, h2s_ref):
    dis = dis_ref[...]
    acc = p_ref[0, :N, :] + p_ref[1, :N, :] + h1s_ref[...]
    h = jnp.maximum(acc * dis[:, None] + b1_ref[...][None, :], 0.0)
    h2 = jnp.dot(h, w2_ref[...], preferred_element_type=jnp.float32)
    h2s_ref[...] = h2[:, 0] * dis


_tc2 = pl.pallas_call(
    _tc2_body,
    out_shape=jax.ShapeDtypeStruct((N,), jnp.float32),
)


def _tc3_body(q_ref, h2s_ref, dis_ref, b2_ref, out_ref):
    out_ref[...] = (
        dis_ref[...] * (q_ref[0, :N] + q_ref[1, :N] + h2s_ref[...]) + b2_ref[...]
    )


_tc3 = pl.pallas_call(
    _tc3_body,
    out_shape=jax.ShapeDtypeStruct((N,), jnp.float32),
)


def kernel(x, edge_index, W1, b1, W2, b2):
    row2 = edge_index[0].reshape(NW, EPT)
    col2 = edge_index[1].reshape(NW, EPT)
    row3 = row2.reshape(NW, NCH2, K2)
    col3 = col2.reshape(NW, NCH2, K2)
    dummy_tab = jnp.zeros((N,), jnp.float32)

    degp = _sc_degree(col2, dummy_tab, row2)
    h1 = _tc_mm(x, W1)
    h1s, dis = _tc1(degp, h1)
    p = _sc_agg128(h1s, row3, col3)
    h2s = _tc2(p, h1s, dis, b1, W2)
    q = _sc_agg1(h2s, row2, col2)
    out = _tc3(q, h2s, dis, b2)
    return out[:, None]
